# Initial kernel scaffold; baseline (speedup 1.0000x reference)
import jax
import jax.numpy as jnp
from jax.experimental import pallas as pl


def kernel(x, tables):
    raise NotImplementedError("write your pallas kernel here")


# speedup vs baseline: 381.5631x; 381.5631x over previous
import functools

import jax
import jax.numpy as jnp
from jax.experimental import pallas as pl

NFEAT = 56
DIM = 128


def _tc_body(x_ref, tab_ref, out_ref):
    tab = tab_ref[...]
    delta = tab[:, 1, :] - tab[:, 0, :]
    base = jnp.sum(tab[:, 0, :], axis=0)
    xf = x_ref[...].astype(jnp.float32)
    acc = jax.lax.dot_general(
        xf, delta, (((1,), (0,)), ((), ())),
        preferred_element_type=jnp.float32)
    out_ref[...] = acc + base[None, :]


def kernel(x, tables):
    n = x.shape[0]
    blk = 2000
    assert n % blk == 0
    return pl.pallas_call(
        _tc_body,
        grid=(n // blk,),
        in_specs=[
            pl.BlockSpec((blk, NFEAT), lambda i: (i, 0)),
            pl.BlockSpec((NFEAT, 2, DIM), lambda i: (0, 0, 0)),
        ],
        out_specs=pl.BlockSpec((blk, DIM), lambda i: (i, 0)),
        out_shape=jax.ShapeDtypeStruct((n, DIM), jnp.float32),
    )(x, tables)
